# baseline (speedup 1.0000x reference)
import functools
import math

import jax
import jax.numpy as jnp
from jax import lax
from jax.experimental import pallas as pl
from jax.experimental.pallas import tpu as pltpu
from jax.experimental.pallas import tpu_sc as plsc

DIM = 1024
HS = 1024
HL = 4096
TEMP = 1.0
CAP_FACTOR = 1.0

R = 256
HC = 1024
NJ = HL // HC


def _ffn_body(ns_ref, xp_ref, sw1_ref, sw3_ref, sw2_ref,
              lw1_ref, lw3_ref, lw2_ref,
              sb1_ref, sb3_ref, sb2_ref, lb1_ref, lb3_ref, lb2_ref,
              out_ref, acc_ref):
    j = pl.program_id(0)
    i = pl.program_id(1)
    ns = ns_ref[0]
    row0 = i * R
    has_small = row0 < ns
    has_large = (row0 + R) > ns

    @pl.when((j == 0) & has_small)
    def _small():
        x = xp_ref[...]
        a = jnp.dot(x, sw1_ref[...], preferred_element_type=jnp.float32)
        b = jnp.dot(x, sw3_ref[...], preferred_element_type=jnp.float32)
        a = a + sb1_ref[...]
        b = b + sb3_ref[...]
        h = (a * jax.nn.sigmoid(a)) * b
        acc_ref[pl.ds(row0, R), :] = (
            jnp.dot(h, sw2_ref[...], preferred_element_type=jnp.float32)
            + sb2_ref[...])

    @pl.when(has_large)
    def _large():
        x = xp_ref[...]
        a = jnp.dot(x, lw1_ref[...], preferred_element_type=jnp.float32)
        b = jnp.dot(x, lw3_ref[...], preferred_element_type=jnp.float32)
        a = a + lb1_ref[...]
        b = b + lb3_ref[...]
        h = (a * jax.nn.sigmoid(a)) * b
        partial = jnp.dot(h, lw2_ref[...], preferred_element_type=jnp.float32)
        rows = row0 + jax.lax.broadcasted_iota(jnp.int32, (R, 1), 0)
        is_large_row = rows >= ns

        @pl.when(j == 0)
        def _init():
            @pl.when(has_small)
            def _masked():
                acc_ref[pl.ds(row0, R), :] = jnp.where(
                    is_large_row, partial + lb2_ref[...],
                    acc_ref[pl.ds(row0, R), :])

            @pl.when(jnp.logical_not(has_small))
            def _plain():
                acc_ref[pl.ds(row0, R), :] = partial + lb2_ref[...]

        @pl.when((j > 0) & (j < NJ - 1))
        def _acc():
            @pl.when(has_small)
            def _masked():
                cur = acc_ref[pl.ds(row0, R), :]
                acc_ref[pl.ds(row0, R), :] = jnp.where(is_large_row,
                                                       cur + partial, cur)

            @pl.when(jnp.logical_not(has_small))
            def _plain():
                acc_ref[pl.ds(row0, R), :] += partial

        @pl.when(j == NJ - 1)
        def _final():
            cur = acc_ref[pl.ds(row0, R), :]

            @pl.when(has_small)
            def _masked():
                out_ref[...] = jnp.where(is_large_row, cur + partial, cur)

            @pl.when(jnp.logical_not(has_small))
            def _plain():
                out_ref[...] = cur + partial

    @pl.when((j == NJ - 1) & jnp.logical_not(has_large))
    def _emit():
        out_ref[...] = acc_ref[pl.ds(row0, R), :]


def _dual_ffn(ns, xp, s_w1, s_w3, s_w2, l_w1, l_w3, l_w2,
              s_b1, s_b3, s_b2, l_b1, l_b3, l_b2, *, interpret=False):
    tokens = xp.shape[0]
    nr = tokens // R

    def block_has_large(i, ns_ref):
        return (i + 1) * R > ns_ref[0]

    def x_map(j, i, ns_ref):
        return (jnp.where((j == 0) | block_has_large(i, ns_ref), i, 0), 0)

    def lw_cols(j, i, ns_ref):
        return (0, jnp.where(ns_ref[0] < tokens, j, 0))

    def lw2_rows(j, i, ns_ref):
        return (jnp.where(ns_ref[0] < tokens, j, 0), 0)

    def out_map(j, i, ns_ref):
        return (jnp.where(j == NJ - 1, i, 0), 0)

    grid_spec = pltpu.PrefetchScalarGridSpec(
        num_scalar_prefetch=1,
        grid=(NJ, nr),
        in_specs=[
            pl.BlockSpec((R, DIM), x_map),
            pl.BlockSpec((DIM, HS), lambda j, i, ns_ref: (0, 0)),
            pl.BlockSpec((DIM, HS), lambda j, i, ns_ref: (0, 0)),
            pl.BlockSpec((HS, DIM), lambda j, i, ns_ref: (0, 0)),
            pl.BlockSpec((DIM, HC), lw_cols),
            pl.BlockSpec((DIM, HC), lw_cols),
            pl.BlockSpec((HC, DIM), lw2_rows),
            pl.BlockSpec((1, HS), lambda j, i, ns_ref: (0, 0)),
            pl.BlockSpec((1, HS), lambda j, i, ns_ref: (0, 0)),
            pl.BlockSpec((1, DIM), lambda j, i, ns_ref: (0, 0)),
            pl.BlockSpec((1, HC), lw_cols),
            pl.BlockSpec((1, HC), lw_cols),
            pl.BlockSpec((1, DIM), lambda j, i, ns_ref: (0, 0)),
        ],
        out_specs=pl.BlockSpec((R, DIM), out_map),
        scratch_shapes=[pltpu.VMEM((tokens, DIM), jnp.float32)],
    )
    return pl.pallas_call(
        _ffn_body,
        grid_spec=grid_spec,
        out_shape=jax.ShapeDtypeStruct((tokens, DIM), jnp.float32),
        compiler_params=pltpu.CompilerParams(
            dimension_semantics=("arbitrary", "arbitrary"),
            vmem_limit_bytes=100 * 1024 * 1024,
        ),
        interpret=interpret,
    )(ns, xp, s_w1, s_w3, s_w2, l_w1, l_w3, l_w2,
      s_b1.reshape(1, HS), s_b3.reshape(1, HS), s_b2.reshape(1, DIM),
      l_b1.reshape(1, HL), l_b3.reshape(1, HL), l_b2.reshape(1, DIM))


_SC_NW = 32
_SC_CH = 64


def _sc_gather_rows(table, idx):
    tokens, d = table.shape
    per_w = tokens // _SC_NW
    nch = per_w // _SC_CH
    mesh = plsc.VectorSubcoreMesh(core_axis_name="c", subcore_axis_name="s")

    @functools.partial(
        pl.kernel, mesh=mesh,
        out_type=jax.ShapeDtypeStruct((tokens, d), table.dtype),
        scratch_types=[
            pltpu.VMEM((_SC_CH,), jnp.int32),
            pltpu.VMEM((_SC_CH, d), table.dtype),
            pltpu.SemaphoreType.DMA,
        ],
    )
    def k(table_hbm, idx_hbm, out_hbm, idx_v, rows_v, sem):
        wid = lax.axis_index("s") * 2 + lax.axis_index("c")
        for c in range(nch):
            base = wid * per_w + c * _SC_CH
            pltpu.sync_copy(idx_hbm.at[pl.ds(base, _SC_CH)], idx_v)
            pltpu.async_copy(table_hbm.at[idx_v], rows_v, sem).wait()
            pltpu.sync_copy(rows_v, out_hbm.at[pl.ds(base, _SC_CH)])

    return k(table, idx)


def _sc_scatter_rows(rows, dest):
    tokens, d = rows.shape
    per_w = tokens // _SC_NW
    nch = per_w // _SC_CH
    mesh = plsc.VectorSubcoreMesh(core_axis_name="c", subcore_axis_name="s")

    @functools.partial(
        pl.kernel, mesh=mesh,
        out_type=jax.ShapeDtypeStruct((tokens, d), rows.dtype),
        scratch_types=[
            pltpu.VMEM((_SC_CH,), jnp.int32),
            pltpu.VMEM((_SC_CH, d), rows.dtype),
            pltpu.SemaphoreType.DMA,
        ],
    )
    def k(rows_hbm, dest_hbm, out_hbm, idx_v, rows_v, sem):
        wid = lax.axis_index("s") * 2 + lax.axis_index("c")
        for c in range(nch):
            base = wid * per_w + c * _SC_CH
            pltpu.sync_copy(dest_hbm.at[pl.ds(base, _SC_CH)], idx_v)
            pltpu.sync_copy(rows_hbm.at[pl.ds(base, _SC_CH)], rows_v)
            pltpu.async_copy(rows_v, out_hbm.at[idx_v], sem).wait()

    return k(rows, dest)


def kernel(x, router_w, router_b, s_w1, s_b1, s_w3, s_b3, s_w2, s_b2,
           l_w1, l_b1, l_w3, l_b3, l_w2, l_b2):
    bsz, seq_len, dim = x.shape
    tokens = bsz * seq_len
    flat = x.reshape(tokens, dim)

    logits = (flat @ router_w + router_b) / max(TEMP, 1e-6)
    large_mask = logits[:, 1] > logits[:, 0]
    capacity = max(1, int(math.ceil(tokens / 2 * CAP_FACTOR)))
    csl = jnp.cumsum(large_mask.astype(jnp.int32))
    cse = jnp.minimum(csl, capacity)
    eff_large = large_mask & (csl <= capacity)
    n_large = cse[-1]
    ns = tokens - n_large

    iota = jnp.arange(tokens, dtype=jnp.int32)
    dest = jnp.where(eff_large, ns + cse - 1, iota - cse)

    xp = _sc_scatter_rows(flat, dest)
    yp = _dual_ffn(ns.reshape(1), xp, s_w1, s_w3, s_w2, l_w1, l_w3, l_w2,
                   s_b1, s_b3, s_b2, l_b1, l_b3, l_b2)
    out = _sc_gather_rows(yp, dest)

    stats = jnp.stack([ns, n_large, jnp.int32(0)])
    return out.reshape(bsz, seq_len, dim), stats


# speedup vs baseline: 1.0719x; 1.0719x over previous
import functools
import math

import jax
import jax.numpy as jnp
from jax import lax
from jax.experimental import pallas as pl
from jax.experimental.pallas import tpu as pltpu
from jax.experimental.pallas import tpu_sc as plsc

DIM = 1024
HS = 1024
HL = 4096
TEMP = 1.0
CAP_FACTOR = 1.0

R = 512
HC = 1024
NJ = HL // HC


def _ffn_body(ns_ref, xp_ref, sw1_ref, sw3_ref, sw2_ref,
              lw1_ref, lw3_ref, lw2_ref,
              sb1_ref, sb3_ref, sb2_ref, lb1_ref, lb3_ref, lb2_ref,
              out_ref, acc_ref):
    j = pl.program_id(0)
    i = pl.program_id(1)
    ns = ns_ref[0]
    row0 = i * R
    has_small = row0 < ns
    has_large = (row0 + R) > ns

    @pl.when((j == 0) & has_small)
    def _small():
        x = xp_ref[...]
        a = jnp.dot(x, sw1_ref[...], preferred_element_type=jnp.float32)
        b = jnp.dot(x, sw3_ref[...], preferred_element_type=jnp.float32)
        a = a + sb1_ref[...]
        b = b + sb3_ref[...]
        h = (a * jax.nn.sigmoid(a)) * b
        acc_ref[pl.ds(row0, R), :] = (
            jnp.dot(h, sw2_ref[...], preferred_element_type=jnp.float32)
            + sb2_ref[...])

    @pl.when(has_large)
    def _large():
        x = xp_ref[...]
        a = jnp.dot(x, lw1_ref[...], preferred_element_type=jnp.float32)
        b = jnp.dot(x, lw3_ref[...], preferred_element_type=jnp.float32)
        a = a + lb1_ref[...]
        b = b + lb3_ref[...]
        h = (a * jax.nn.sigmoid(a)) * b
        partial = jnp.dot(h, lw2_ref[...], preferred_element_type=jnp.float32)
        rows = row0 + jax.lax.broadcasted_iota(jnp.int32, (R, 1), 0)
        is_large_row = rows >= ns

        @pl.when(j == 0)
        def _init():
            @pl.when(has_small)
            def _masked():
                acc_ref[pl.ds(row0, R), :] = jnp.where(
                    is_large_row, partial + lb2_ref[...],
                    acc_ref[pl.ds(row0, R), :])

            @pl.when(jnp.logical_not(has_small))
            def _plain():
                acc_ref[pl.ds(row0, R), :] = partial + lb2_ref[...]

        @pl.when(j > 0)
        def _acc():
            @pl.when(has_small)
            def _masked():
                cur = acc_ref[pl.ds(row0, R), :]
                acc_ref[pl.ds(row0, R), :] = jnp.where(is_large_row,
                                                       cur + partial, cur)

            @pl.when(jnp.logical_not(has_small))
            def _plain():
                acc_ref[pl.ds(row0, R), :] += partial

    @pl.when(j == NJ - 1)
    def _emit():
        out_ref[...] = acc_ref[pl.ds(row0, R), :]


def _dual_ffn(ns, xp, s_w1, s_w3, s_w2, l_w1, l_w3, l_w2,
              s_b1, s_b3, s_b2, l_b1, l_b3, l_b2, *, interpret=False):
    tokens = xp.shape[0]
    nr = tokens // R

    def block_has_large(i, ns_ref):
        return (i + 1) * R > ns_ref[0]

    def x_map(j, i, ns_ref):
        return (jnp.where((j == 0) | block_has_large(i, ns_ref), i, 0), 0)

    def lw_cols(j, i, ns_ref):
        return (0, jnp.where(ns_ref[0] < tokens, j, 0))

    def lw2_rows(j, i, ns_ref):
        return (jnp.where(ns_ref[0] < tokens, j, 0), 0)

    def out_map(j, i, ns_ref):
        return (jnp.where(j == NJ - 1, i, 0), 0)

    grid_spec = pltpu.PrefetchScalarGridSpec(
        num_scalar_prefetch=1,
        grid=(NJ, nr),
        in_specs=[
            pl.BlockSpec((R, DIM), x_map),
            pl.BlockSpec((DIM, HS), lambda j, i, ns_ref: (0, 0)),
            pl.BlockSpec((DIM, HS), lambda j, i, ns_ref: (0, 0)),
            pl.BlockSpec((HS, DIM), lambda j, i, ns_ref: (0, 0)),
            pl.BlockSpec((DIM, HC), lw_cols),
            pl.BlockSpec((DIM, HC), lw_cols),
            pl.BlockSpec((HC, DIM), lw2_rows),
            pl.BlockSpec((1, HS), lambda j, i, ns_ref: (0, 0)),
            pl.BlockSpec((1, HS), lambda j, i, ns_ref: (0, 0)),
            pl.BlockSpec((1, DIM), lambda j, i, ns_ref: (0, 0)),
            pl.BlockSpec((1, HC), lw_cols),
            pl.BlockSpec((1, HC), lw_cols),
            pl.BlockSpec((1, DIM), lambda j, i, ns_ref: (0, 0)),
        ],
        out_specs=pl.BlockSpec((R, DIM), out_map),
        scratch_shapes=[pltpu.VMEM((tokens, DIM), jnp.float32)],
    )
    return pl.pallas_call(
        _ffn_body,
        grid_spec=grid_spec,
        out_shape=jax.ShapeDtypeStruct((tokens, DIM), jnp.float32),
        compiler_params=pltpu.CompilerParams(
            dimension_semantics=("arbitrary", "arbitrary"),
            vmem_limit_bytes=100 * 1024 * 1024,
        ),
        interpret=interpret,
    )(ns, xp, s_w1, s_w3, s_w2, l_w1, l_w3, l_w2,
      s_b1.reshape(1, HS), s_b3.reshape(1, HS), s_b2.reshape(1, DIM),
      l_b1.reshape(1, HL), l_b3.reshape(1, HL), l_b2.reshape(1, DIM))


_SC_NW = 32
_SC_CH = 64


def _sc_gather_rows(table, idx):
    tokens, d = table.shape
    per_w = tokens // _SC_NW
    nch = per_w // _SC_CH
    mesh = plsc.VectorSubcoreMesh(core_axis_name="c", subcore_axis_name="s")

    @functools.partial(
        pl.kernel, mesh=mesh,
        out_type=jax.ShapeDtypeStruct((tokens, d), table.dtype),
        scratch_types=[
            pltpu.VMEM((_SC_CH,), jnp.int32),
            pltpu.VMEM((_SC_CH, d), table.dtype),
            pltpu.SemaphoreType.DMA,
        ],
    )
    def k(table_hbm, idx_hbm, out_hbm, idx_v, rows_v, sem):
        wid = lax.axis_index("s") * 2 + lax.axis_index("c")
        for c in range(nch):
            base = wid * per_w + c * _SC_CH
            pltpu.sync_copy(idx_hbm.at[pl.ds(base, _SC_CH)], idx_v)
            pltpu.async_copy(table_hbm.at[idx_v], rows_v, sem).wait()
            pltpu.sync_copy(rows_v, out_hbm.at[pl.ds(base, _SC_CH)])

    return k(table, idx)


def _sc_scatter_rows(rows, dest):
    tokens, d = rows.shape
    per_w = tokens // _SC_NW
    nch = per_w // _SC_CH
    mesh = plsc.VectorSubcoreMesh(core_axis_name="c", subcore_axis_name="s")

    @functools.partial(
        pl.kernel, mesh=mesh,
        out_type=jax.ShapeDtypeStruct((tokens, d), rows.dtype),
        scratch_types=[
            pltpu.VMEM((_SC_CH,), jnp.int32),
            pltpu.VMEM((_SC_CH, d), rows.dtype),
            pltpu.SemaphoreType.DMA,
        ],
    )
    def k(rows_hbm, dest_hbm, out_hbm, idx_v, rows_v, sem):
        wid = lax.axis_index("s") * 2 + lax.axis_index("c")
        for c in range(nch):
            base = wid * per_w + c * _SC_CH
            pltpu.sync_copy(dest_hbm.at[pl.ds(base, _SC_CH)], idx_v)
            pltpu.sync_copy(rows_hbm.at[pl.ds(base, _SC_CH)], rows_v)
            pltpu.async_copy(rows_v, out_hbm.at[idx_v], sem).wait()

    return k(rows, dest)


def kernel(x, router_w, router_b, s_w1, s_b1, s_w3, s_b3, s_w2, s_b2,
           l_w1, l_b1, l_w3, l_b3, l_w2, l_b2):
    bsz, seq_len, dim = x.shape
    tokens = bsz * seq_len
    flat = x.reshape(tokens, dim)

    logits = (flat @ router_w + router_b) / max(TEMP, 1e-6)
    large_mask = logits[:, 1] > logits[:, 0]
    capacity = max(1, int(math.ceil(tokens / 2 * CAP_FACTOR)))
    csl = jnp.cumsum(large_mask.astype(jnp.int32))
    cse = jnp.minimum(csl, capacity)
    eff_large = large_mask & (csl <= capacity)
    n_large = cse[-1]
    ns = tokens - n_large

    iota = jnp.arange(tokens, dtype=jnp.int32)
    dest = jnp.where(eff_large, ns + cse - 1, iota - cse)

    xp = _sc_scatter_rows(flat, dest)
    yp = _dual_ffn(ns.reshape(1), xp, s_w1, s_w3, s_w2, l_w1, l_w3, l_w2,
                   s_b1, s_b3, s_b2, l_b1, l_b3, l_b2)
    out = _sc_gather_rows(yp, dest)

    stats = jnp.stack([ns, n_large, jnp.int32(0)])
    return out.reshape(bsz, seq_len, dim), stats
